# baseline (device time: 22922 ns/iter reference)
import jax
import jax.numpy as jnp
from jax import lax
from jax.experimental import pallas as pl
from jax.experimental.pallas import tpu as pltpu

Y = 4
Z = 4
M_OUT = 128
D = 512
N = 2048
CW = N // Z
H = 2
HW = CW // H


def kernel(x, dy):
    def body(x_ref, dy_ref, out_ref, part_ref, rs_buf, ag_buf,
             rs_send, rs_recv, ag_send, ag_recv):
        my_x = lax.axis_index("x")
        my_y = lax.axis_index("y")
        my_z = lax.axis_index("z")

        barrier_sem = pltpu.get_barrier_semaphore()
        for d in range(1, Y):
            pl.semaphore_signal(
                barrier_sem, inc=1,
                device_id=(my_x, lax.rem(my_y + d, Y), my_z),
                device_id_type=pl.DeviceIdType.MESH,
            )
        for d in range(1, Z):
            pl.semaphore_signal(
                barrier_sem, inc=1,
                device_id=(my_x, my_y, lax.rem(my_z + d, Z)),
                device_id_type=pl.DeviceIdType.MESH,
            )
        pl.semaphore_wait(barrier_sem, (Y - 1) + (Z - 1))

        part = lax.dot_general(
            x_ref[...].astype(jnp.bfloat16),
            dy_ref[:, pl.ds(my_z * CW, CW)].astype(jnp.bfloat16),
            dimension_numbers=(((0,), (0,)), ((), ())),
            preferred_element_type=jnp.float32,
        )
        part_ref[...] = part.astype(jnp.bfloat16)

        def rows(j):
            return pl.ds(j * M_OUT, M_OUT)

        def half(h):
            return pl.ds(h * HW, HW)

        rs_rdmas = [[None] * (Y - 1) for _ in range(H)]
        for h in range(H):
            for d in range(Y - 1, 0, -1):
                tgt = lax.rem(my_y + d, Y)
                rdma = pltpu.make_async_remote_copy(
                    src_ref=part_ref.at[rows(tgt), half(h)],
                    dst_ref=rs_buf.at[h, d - 1],
                    send_sem=rs_send.at[h * (Y - 1) + d - 1],
                    recv_sem=rs_recv.at[h * (Y - 1) + d - 1],
                    device_id=(my_x, tgt, my_z),
                    device_id_type=pl.DeviceIdType.MESH,
                )
                rdma.start()
                rs_rdmas[h][d - 1] = rdma

        ag_rdmas = [[None] * (Z - 1) for _ in range(H)]
        for h in range(H):
            for d in range(1, Y):
                rs_rdmas[h][d - 1].wait_recv()
            acc = (
                part_ref[rows(my_y), half(h)].astype(jnp.float32)
                + rs_buf[h, 0].astype(jnp.float32)
                + rs_buf[h, 1].astype(jnp.float32)
                + rs_buf[h, 2].astype(jnp.float32)
            )
            out_ref[:, pl.ds(my_z * CW + h * HW, HW)] = acc
            ag_buf[h, Z - 1, :, :] = acc.astype(jnp.bfloat16)
            for d in range(Z - 1, 0, -1):
                tgt = lax.rem(my_z + d, Z)
                rdma = pltpu.make_async_remote_copy(
                    src_ref=ag_buf.at[h, Z - 1],
                    dst_ref=ag_buf.at[h, d - 1],
                    send_sem=ag_send.at[h * (Z - 1) + d - 1],
                    recv_sem=ag_recv.at[h * (Z - 1) + d - 1],
                    device_id=(my_x, my_y, tgt),
                    device_id_type=pl.DeviceIdType.MESH,
                )
                rdma.start()
                ag_rdmas[h][d - 1] = rdma

        for h in range(H):
            for d in range(1, Z):
                ag_rdmas[h][d - 1].wait_recv()
                src_z = lax.rem(my_z + Z - d, Z)
                out_ref[:, pl.ds(src_z * CW + h * HW, HW)] = (
                    ag_buf[h, d - 1].astype(jnp.float32)
                )

        for h in range(H):
            for rdma in rs_rdmas[h] + ag_rdmas[h]:
                rdma.wait_send()

    return pl.pallas_call(
        body,
        out_shape=jax.ShapeDtypeStruct((M_OUT, N), jnp.float32),
        in_specs=[
            pl.BlockSpec(memory_space=pltpu.VMEM),
            pl.BlockSpec(memory_space=pltpu.VMEM),
        ],
        out_specs=pl.BlockSpec(memory_space=pltpu.VMEM),
        scratch_shapes=[
            pltpu.VMEM((D, CW), jnp.bfloat16),
            pltpu.VMEM((H, Y - 1, M_OUT, HW), jnp.bfloat16),
            pltpu.VMEM((H, Z, M_OUT, HW), jnp.bfloat16),
            pltpu.SemaphoreType.DMA((H * (Y - 1),)),
            pltpu.SemaphoreType.DMA((H * (Y - 1),)),
            pltpu.SemaphoreType.DMA((H * (Z - 1),)),
            pltpu.SemaphoreType.DMA((H * (Z - 1),)),
        ],
        compiler_params=pltpu.CompilerParams(collective_id=0),
    )(x, dy)
